# baseline (speedup 1.0000x reference)
import jax
import jax.numpy as jnp
from jax import lax
from jax.experimental import pallas as pl
from jax.experimental.pallas import tpu as pltpu
from jax.experimental.pallas import tpu_sc as plsc

N_NODES = 10000
N_GRAPHS = 64
N_TILES = 16
ROWS_MAIN = 632
ROWS_LAST = N_NODES - (N_TILES - 1) * ROWS_MAIN
CW = 128
F32 = jnp.float32

_MESH = plsc.VectorSubcoreMesh(core_axis_name="c", subcore_axis_name="s")
_SC_PARAMS = pltpu.CompilerParams(use_tc_tiling_on_sc=False)

_PREC = jax.lax.Precision.DEFAULT


def _dotT(a, b):
    return jax.lax.dot_general(
        a, b, dimension_numbers=(((1,), (1,)), ((), ())),
        precision=_PREC, preferred_element_type=F32)


def _make_sc_segsum(n_chunks: int, n_batches: int, edge_batch: int):
    cc = n_chunks // 2
    nb = n_batches

    scratch = [
        pltpu.VMEM((2, edge_batch), jnp.int32),
        pltpu.VMEM((2, edge_batch), jnp.int32),
        pltpu.VMEM((2, edge_batch), jnp.int32),
        pltpu.VMEM((edge_batch, CW), F32),
        pltpu.VMEM((edge_batch, CW), F32),
        pltpu.VMEM_SHARED((N_NODES + 8, CW), F32),
        pltpu.SemaphoreType.DMA,
        pltpu.SemaphoreType.DMA,
        pltpu.SemaphoreType.DMA,
        pltpu.SemaphoreType.DMA,
        pltpu.SemaphoreType.DMA,
        pltpu.SemaphoreType.DMA,
        pltpu.SemaphoreType.DMA,
    ]

    def body(table, idx_h, zrow_h, out_h,
             ib0, ib1, ib2, rows0_v, rows1_v, acc,
             isem0, isem1, isem2, gsem0, gsem1, ssem0, ssem1):
        ibs = (ib0, ib1, ib2)
        isems = (isem0, isem1, isem2)
        rbs = (rows0_v, rows1_v)
        gsems = (gsem0, gsem1)
        ssems = (ssem0, ssem1)

        cid = lax.axis_index("c")
        sid = lax.axis_index("s")

        def per_tile_rows(fn):
            @pl.when(sid < N_TILES - 1)
            def _():
                fn(sid * ROWS_MAIN, ROWS_MAIN)

            @pl.when(sid == N_TILES - 1)
            def _():
                fn((N_TILES - 1) * ROWS_MAIN, ROWS_LAST)

        def idx_load(b):
            return pltpu.async_copy(idx_h.at[sid, b], ibs[b % 3], isems[b % 3])

        for j in range(cc):
            c = cid * cc + j
            per_tile_rows(lambda r0, nr: pltpu.sync_copy(
                zrow_h.at[pl.ds(r0, nr)], acc.at[pl.ds(r0, nr)]))
            plsc.subcore_barrier()

            tbl_c = table.at[c]

            def gather(b):
                return pltpu.async_copy(
                    tbl_c.at[ibs[b % 3].at[0]], rbs[b % 2], gsems[b % 2])

            def scatter(b):
                return pltpu.async_copy(
                    rbs[b % 2], acc.at[ibs[b % 3].at[1]], ssems[b % 2],
                    add=True)

            il = {b: idx_load(b) for b in range(min(3, nb))}
            il[0].wait()
            g = {0: gather(0)}
            s = {}
            s_waited = set()
            for b in range(nb):
                g[b].wait()
                if b + 1 < nb:
                    il[b + 1].wait()
                    if b - 1 in s:
                        s[b - 1].wait()
                        s_waited.add(b - 1)
                        if b + 2 < nb and b + 2 >= 3:
                            il[b + 2] = idx_load(b + 2)
                    g[b + 1] = gather(b + 1)
                s[b] = scatter(b)
            for b in range(nb):
                if b in s and b not in s_waited:
                    s[b].wait()
            plsc.subcore_barrier()

            per_tile_rows(lambda r0, nr: pltpu.sync_copy(
                acc.at[pl.ds(r0, nr)], out_h.at[c, pl.ds(r0, nr)]))

    return pl.kernel(
        body,
        out_type=jax.ShapeDtypeStruct((n_chunks, N_NODES, CW), F32),
        mesh=_MESH, scratch_types=scratch, compiler_params=_SC_PARAMS)


def _make_sc_cnt(n_batches: int, edge_batch: int):
    scratch = [
        pltpu.VMEM((n_batches, edge_batch), jnp.int32),
        pltpu.VMEM((edge_batch, 16), F32),
        pltpu.VMEM_SHARED((N_NODES, 16), F32),
        pltpu.SemaphoreType.DMA,
    ]

    def body(dst_h, ones_h, z16_h, cnt_h, dst_v, ones_v, cntacc, csem):
        cid = lax.axis_index("c")
        sid = lax.axis_index("s")

        def per_tile_rows(fn):
            @pl.when(sid < N_TILES - 1)
            def _():
                fn(sid * ROWS_MAIN, ROWS_MAIN)

            @pl.when(sid == N_TILES - 1)
            def _():
                fn((N_TILES - 1) * ROWS_MAIN, ROWS_LAST)

        pltpu.sync_copy(dst_h.at[cid, sid], dst_v)
        pltpu.sync_copy(ones_h, ones_v)
        per_tile_rows(lambda r0, nr: pltpu.sync_copy(
            z16_h.at[pl.ds(0, nr)], cntacc.at[pl.ds(r0, nr)]))
        plsc.subcore_barrier()
        pending = []
        for b in range(n_batches):
            pending.append(pltpu.async_copy(
                ones_v, cntacc.at[dst_v.at[b]], csem, add=True))
        for p in pending:
            p.wait()
        plsc.subcore_barrier()
        per_tile_rows(lambda r0, nr: pltpu.sync_copy(
            cntacc.at[pl.ds(r0, nr)], cnt_h.at[cid, pl.ds(r0, nr)]))

    return pl.kernel(
        body,
        out_type=jax.ShapeDtypeStruct((2, N_NODES, 16), F32),
        mesh=_MESH, scratch_types=scratch, compiler_params=_SC_PARAMS)


_BLK = 1000
_NBLK = N_NODES // _BLK


def _inv_from_cnt(cnt_ref):
    c = cnt_ref[0, :, :1] + cnt_ref[1, :, :1]
    return 1.0 / jnp.maximum(c, 1.0)


def _tc_chunk(x):
    d_in = x.shape[1]
    nc = d_in // CW

    def body(x_ref, xck_ref):
        xb = x_ref[...]
        for c in range(nc):
            xck_ref[c] = xb[:, c * CW:(c + 1) * CW]

    return pl.pallas_call(
        body,
        grid=(_NBLK,),
        in_specs=[pl.BlockSpec((_BLK, d_in), lambda i: (i, 0))],
        out_specs=pl.BlockSpec((nc, _BLK, CW), lambda i: (0, i, 0)),
        out_shape=jax.ShapeDtypeStruct((nc, N_NODES, CW), F32),
    )(x)


def _tc_r(hck, wn):
    nc = hck.shape[0]
    d_in = nc * CW

    def body(hck_ref, wn_ref, rn_ref):
        hb = jnp.concatenate([hck_ref[c] for c in range(nc)], axis=1)
        rn_ref[...] = _dotT(hb, wn_ref[...])

    return pl.pallas_call(
        body,
        grid=(_NBLK,),
        in_specs=[
            pl.BlockSpec((nc, _BLK, CW), lambda i: (0, i, 0)),
            pl.BlockSpec((512, d_in), lambda i: (0, 0)),
        ],
        out_specs=pl.BlockSpec((_BLK, 512), lambda i: (i, 0)),
        out_shape=jax.ShapeDtypeStruct((N_NODES, 512), F32),
    )(hck, wn)


def _tc_layer(aggc, cnt, r, wl, bl, w_next, relu: bool):
    nc_in = aggc.shape[0]
    d_in = nc_in * CW

    def body(aggc_ref, cnt_ref, r_ref, wl_ref, bl_ref, wn_ref,
             hck_ref, rn_ref):
        agg = jnp.concatenate([aggc_ref[c] for c in range(nc_in)], axis=1)
        h = _dotT(agg * _inv_from_cnt(cnt_ref), wl_ref[...]) \
            + bl_ref[...] + r_ref[...]
        if relu:
            h = jnp.maximum(h, 0.0)
        rn_ref[...] = _dotT(h, wn_ref[...])
        for c in range(4):
            hck_ref[c] = h[:, c * CW:(c + 1) * CW]

    return pl.pallas_call(
        body,
        grid=(_NBLK,),
        in_specs=[
            pl.BlockSpec((nc_in, _BLK, CW), lambda i: (0, i, 0)),
            pl.BlockSpec((2, _BLK, 16), lambda i: (0, i, 0)),
            pl.BlockSpec((_BLK, 512), lambda i: (i, 0)),
            pl.BlockSpec((512, d_in), lambda i: (0, 0)),
            pl.BlockSpec((1, 512), lambda i: (0, 0)),
            pl.BlockSpec((512, 512), lambda i: (0, 0)),
        ],
        out_specs=[
            pl.BlockSpec((4, _BLK, CW), lambda i: (0, i, 0)),
            pl.BlockSpec((_BLK, 512), lambda i: (i, 0)),
        ],
        out_shape=[
            jax.ShapeDtypeStruct((4, N_NODES, CW), F32),
            jax.ShapeDtypeStruct((N_NODES, 512), F32),
        ],
    )(aggc, cnt, r, wl, bl, w_next)


def _tc_final(aggc, cnt, r, wl, bl, batch2d, wlin, blin):
    def body(aggc_ref, cnt_ref, r_ref, wl_ref, bl_ref, b_ref,
             wlin_ref, blin_ref, out_ref, ps_acc, gc_acc):
        i = pl.program_id(0)

        @pl.when(i == 0)
        def _():
            ps_acc[...] = jnp.zeros_like(ps_acc)
            gc_acc[...] = jnp.zeros_like(gc_acc)

        agg = jnp.concatenate([aggc_ref[c] for c in range(4)], axis=1)
        h = _dotT(agg * _inv_from_cnt(cnt_ref), wl_ref[...]) \
            + bl_ref[...] + r_ref[...]

        gids = jax.lax.broadcasted_iota(jnp.int32, (1, N_GRAPHS), 1)
        onehot = (b_ref[...] == gids).astype(F32)
        ps_acc[...] += jax.lax.dot_general(
            onehot, h, dimension_numbers=(((0,), (0,)), ((), ())),
            precision=_PREC, preferred_element_type=F32)
        gc_acc[...] += jnp.sum(onehot, axis=0)[:, None]

        @pl.when(i == _NBLK - 1)
        def _():
            pooled = ps_acc[...] / jnp.maximum(gc_acc[...], 1.0)
            out_ref[...] = _dotT(pooled, wlin_ref[...]) + blin_ref[...]

    return pl.pallas_call(
        body,
        grid=(_NBLK,),
        in_specs=[
            pl.BlockSpec((4, _BLK, CW), lambda i: (0, i, 0)),
            pl.BlockSpec((2, _BLK, 16), lambda i: (0, i, 0)),
            pl.BlockSpec((_BLK, 512), lambda i: (i, 0)),
            pl.BlockSpec((512, 512), lambda i: (0, 0)),
            pl.BlockSpec((1, 512), lambda i: (0, 0)),
            pl.BlockSpec((_BLK, 1), lambda i: (i, 0)),
            pl.BlockSpec((N_GRAPHS, 512), lambda i: (0, 0)),
            pl.BlockSpec((1, N_GRAPHS), lambda i: (0, 0)),
        ],
        out_specs=pl.BlockSpec((N_GRAPHS, N_GRAPHS), lambda i: (0, 0)),
        out_shape=jax.ShapeDtypeStruct((N_GRAPHS, N_GRAPHS), F32),
        scratch_shapes=[
            pltpu.VMEM((N_GRAPHS, 512), F32),
            pltpu.VMEM((N_GRAPHS, 1), F32),
        ],
    )(aggc, cnt, r, wl, bl, batch2d, wlin, blin)


def kernel(x, edge_index, batch, W1l, b1l, W1r, W2l, b2l, W2r, W3l, b3l, W3r,
           Wlin, blin):
    n_edges = edge_index.shape[1]
    eb = 192
    e_tile = n_edges // N_TILES
    nb = -(-e_tile // eb)
    pad = nb * eb - e_tile
    ebc, nbc = 500, n_edges // (2 * N_TILES * 500)
    assert e_tile * N_TILES == n_edges
    assert nbc * 2 * N_TILES * ebc == n_edges

    src32 = edge_index[0].astype(jnp.int32)
    dst32 = edge_index[1].astype(jnp.int32)
    pad_src = jnp.broadcast_to(jnp.arange(pad, dtype=jnp.int32) % 8,
                               (N_TILES, pad))
    pad_dst = pad_src + N_NODES
    src_p = jnp.concatenate(
        [src32.reshape(N_TILES, e_tile), pad_src], axis=1).reshape(
            N_TILES, nb, 1, eb)
    dst_p = jnp.concatenate(
        [dst32.reshape(N_TILES, e_tile), pad_dst], axis=1).reshape(
            N_TILES, nb, 1, eb)
    idx_p = jnp.concatenate([src_p, dst_p], axis=2)
    dstc = dst32.reshape(2, N_TILES, nbc, ebc)
    batch2d = batch.astype(jnp.int32).reshape(N_NODES, 1)
    zrow = jnp.zeros((N_NODES, CW), F32)
    z16 = jnp.zeros((ROWS_MAIN, 16), F32)
    ones16 = jnp.ones((ebc, 16), F32)
    b1l2 = b1l.reshape(1, 512)
    b2l2 = b2l.reshape(1, 512)
    b3l2 = b3l.reshape(1, 512)
    blin2 = blin.reshape(1, N_GRAPHS)

    sc_first = _make_sc_segsum(2, nb, eb)
    sc_hidden = _make_sc_segsum(4, nb, eb)
    sc_cnt = _make_sc_cnt(nbc, ebc)

    cnt = sc_cnt(dstc, ones16, z16)
    xck = _tc_chunk(x)
    agg1c = sc_first(xck, idx_p, zrow)
    r1 = _tc_r(xck, W1r)
    h1ck, r2 = _tc_layer(agg1c, cnt, r1, W1l, b1l2, W2r, relu=True)
    agg2c = sc_hidden(h1ck, idx_p, zrow)
    h2ck, r3 = _tc_layer(agg2c, cnt, r2, W2l, b2l2, W3r, relu=True)
    agg3c = sc_hidden(h2ck, idx_p, zrow)
    out = _tc_final(agg3c, cnt, r3, W3l, b3l2, batch2d, Wlin, blin2)
    return out


# speedup vs baseline: 1.0239x; 1.0239x over previous
import jax
import jax.numpy as jnp
from jax import lax
from jax.experimental import pallas as pl
from jax.experimental.pallas import tpu as pltpu
from jax.experimental.pallas import tpu_sc as plsc

N_NODES = 10000
N_GRAPHS = 64
N_TILES = 16
ROWS_MAIN = 632
ROWS_LAST = N_NODES - (N_TILES - 1) * ROWS_MAIN
CW = 128
F32 = jnp.float32

_MESH = plsc.VectorSubcoreMesh(core_axis_name="c", subcore_axis_name="s")
_SC_PARAMS = pltpu.CompilerParams(use_tc_tiling_on_sc=False)

_PREC = jax.lax.Precision.DEFAULT


def _dotT(a, b):
    return jax.lax.dot_general(
        a, b, dimension_numbers=(((1,), (1,)), ((), ())),
        precision=_PREC, preferred_element_type=F32)


def _make_sc_segsum(n_chunks: int, n_batches: int, edge_batch: int):
    cc = n_chunks // 2
    nb = n_batches

    scratch = [
        pltpu.VMEM((2, edge_batch), jnp.int32),
        pltpu.VMEM((2, edge_batch), jnp.int32),
        pltpu.VMEM((2, edge_batch), jnp.int32),
        pltpu.VMEM((edge_batch, CW), F32),
        pltpu.VMEM((edge_batch, CW), F32),
        pltpu.VMEM_SHARED((N_NODES + 8, CW), F32),
        pltpu.SemaphoreType.DMA,
        pltpu.SemaphoreType.DMA,
        pltpu.SemaphoreType.DMA,
        pltpu.SemaphoreType.DMA,
        pltpu.SemaphoreType.DMA,
        pltpu.SemaphoreType.DMA,
        pltpu.SemaphoreType.DMA,
    ]

    def body(table, idx_h, zrow_h, out_h,
             ib0, ib1, ib2, rows0_v, rows1_v, acc,
             isem0, isem1, isem2, gsem0, gsem1, ssem0, ssem1):
        ibs = (ib0, ib1, ib2)
        isems = (isem0, isem1, isem2)
        rbs = (rows0_v, rows1_v)
        gsems = (gsem0, gsem1)
        ssems = (ssem0, ssem1)

        cid = lax.axis_index("c")
        sid = lax.axis_index("s")

        def per_tile_rows(fn):
            @pl.when(sid < N_TILES - 1)
            def _():
                fn(sid * ROWS_MAIN, ROWS_MAIN)

            @pl.when(sid == N_TILES - 1)
            def _():
                fn((N_TILES - 1) * ROWS_MAIN, ROWS_LAST)

        def idx_load(b):
            return pltpu.async_copy(idx_h.at[sid, b], ibs[b % 3], isems[b % 3])

        for j in range(cc):
            c = cid * cc + j
            per_tile_rows(lambda r0, nr: pltpu.sync_copy(
                zrow_h.at[pl.ds(0, nr)], acc.at[pl.ds(r0, nr)]))
            plsc.subcore_barrier()

            tbl_c = table.at[c]

            def gather(b):
                return pltpu.async_copy(
                    tbl_c.at[ibs[b % 3].at[0]], rbs[b % 2], gsems[b % 2])

            def scatter(b):
                return pltpu.async_copy(
                    rbs[b % 2], acc.at[ibs[b % 3].at[1]], ssems[b % 2],
                    add=True)

            il = {b: idx_load(b) for b in range(min(3, nb))}
            il[0].wait()
            g = {0: gather(0)}
            s = {}
            s_waited = set()
            for b in range(nb):
                g[b].wait()
                if b + 1 < nb:
                    il[b + 1].wait()
                    if b - 1 in s:
                        s[b - 1].wait()
                        s_waited.add(b - 1)
                        if b + 2 < nb and b + 2 >= 3:
                            il[b + 2] = idx_load(b + 2)
                    g[b + 1] = gather(b + 1)
                s[b] = scatter(b)
            for b in range(nb):
                if b in s and b not in s_waited:
                    s[b].wait()
            plsc.subcore_barrier()

            per_tile_rows(lambda r0, nr: pltpu.sync_copy(
                acc.at[pl.ds(r0, nr)], out_h.at[c, pl.ds(r0, nr)]))

    return pl.kernel(
        body,
        out_type=jax.ShapeDtypeStruct((n_chunks, N_NODES, CW), F32),
        mesh=_MESH, scratch_types=scratch, compiler_params=_SC_PARAMS)


def _make_sc_cnt(n_batches: int, edge_batch: int):
    scratch = [
        pltpu.VMEM((n_batches, edge_batch), jnp.int32),
        pltpu.VMEM((edge_batch, 16), F32),
        pltpu.VMEM_SHARED((N_NODES, 16), F32),
        pltpu.SemaphoreType.DMA,
    ]

    def body(dst_h, ones_h, z16_h, cnt_h, dst_v, ones_v, cntacc, csem):
        cid = lax.axis_index("c")
        sid = lax.axis_index("s")

        def per_tile_rows(fn):
            @pl.when(sid < N_TILES - 1)
            def _():
                fn(sid * ROWS_MAIN, ROWS_MAIN)

            @pl.when(sid == N_TILES - 1)
            def _():
                fn((N_TILES - 1) * ROWS_MAIN, ROWS_LAST)

        pltpu.sync_copy(dst_h.at[cid, sid], dst_v)
        pltpu.sync_copy(ones_h, ones_v)
        per_tile_rows(lambda r0, nr: pltpu.sync_copy(
            z16_h.at[pl.ds(0, nr)], cntacc.at[pl.ds(r0, nr)]))
        plsc.subcore_barrier()
        pending = []
        for b in range(n_batches):
            pending.append(pltpu.async_copy(
                ones_v, cntacc.at[dst_v.at[b]], csem, add=True))
        for p in pending:
            p.wait()
        plsc.subcore_barrier()
        per_tile_rows(lambda r0, nr: pltpu.sync_copy(
            cntacc.at[pl.ds(r0, nr)], cnt_h.at[cid, pl.ds(r0, nr)]))

    return pl.kernel(
        body,
        out_type=jax.ShapeDtypeStruct((2, N_NODES, 16), F32),
        mesh=_MESH, scratch_types=scratch, compiler_params=_SC_PARAMS)


_BLK = 1000
_NBLK = N_NODES // _BLK


def _inv_from_cnt(cnt_ref):
    c = cnt_ref[0, :, :1] + cnt_ref[1, :, :1]
    return 1.0 / jnp.maximum(c, 1.0)


def _tc_prep(x, w1r):
    d_in = x.shape[1]
    nc = d_in // CW

    def body(x_ref, w_ref, xck_ref, r_ref):
        xb = x_ref[...]
        r_ref[...] = _dotT(xb, w_ref[...])
        for c in range(nc):
            xck_ref[c] = xb[:, c * CW:(c + 1) * CW]

    return pl.pallas_call(
        body,
        grid=(_NBLK,),
        in_specs=[
            pl.BlockSpec((_BLK, d_in), lambda i: (i, 0)),
            pl.BlockSpec((512, d_in), lambda i: (0, 0)),
        ],
        out_specs=[
            pl.BlockSpec((nc, _BLK, CW), lambda i: (0, i, 0)),
            pl.BlockSpec((_BLK, 512), lambda i: (i, 0)),
        ],
        out_shape=[
            jax.ShapeDtypeStruct((nc, N_NODES, CW), F32),
            jax.ShapeDtypeStruct((N_NODES, 512), F32),
        ],
    )(x, w1r)


def _tc_layer(aggc, cnt, r, wl, bl, w_next, relu: bool):
    nc_in = aggc.shape[0]
    d_in = nc_in * CW

    def body(aggc_ref, cnt_ref, r_ref, wl_ref, bl_ref, wn_ref,
             hck_ref, rn_ref):
        agg = jnp.concatenate([aggc_ref[c] for c in range(nc_in)], axis=1)
        h = _dotT(agg * _inv_from_cnt(cnt_ref), wl_ref[...]) \
            + bl_ref[...] + r_ref[...]
        if relu:
            h = jnp.maximum(h, 0.0)
        rn_ref[...] = _dotT(h, wn_ref[...])
        for c in range(4):
            hck_ref[c] = h[:, c * CW:(c + 1) * CW]

    return pl.pallas_call(
        body,
        grid=(_NBLK,),
        in_specs=[
            pl.BlockSpec((nc_in, _BLK, CW), lambda i: (0, i, 0)),
            pl.BlockSpec((2, _BLK, 16), lambda i: (0, i, 0)),
            pl.BlockSpec((_BLK, 512), lambda i: (i, 0)),
            pl.BlockSpec((512, d_in), lambda i: (0, 0)),
            pl.BlockSpec((1, 512), lambda i: (0, 0)),
            pl.BlockSpec((512, 512), lambda i: (0, 0)),
        ],
        out_specs=[
            pl.BlockSpec((4, _BLK, CW), lambda i: (0, i, 0)),
            pl.BlockSpec((_BLK, 512), lambda i: (i, 0)),
        ],
        out_shape=[
            jax.ShapeDtypeStruct((4, N_NODES, CW), F32),
            jax.ShapeDtypeStruct((N_NODES, 512), F32),
        ],
    )(aggc, cnt, r, wl, bl, w_next)


def _tc_final(aggc, cnt, r, wl, bl, batch2d, wlin, blin):
    def body(aggc_ref, cnt_ref, r_ref, wl_ref, bl_ref, b_ref,
             wlin_ref, blin_ref, out_ref, ps_acc, gc_acc):
        i = pl.program_id(0)

        @pl.when(i == 0)
        def _():
            ps_acc[...] = jnp.zeros_like(ps_acc)
            gc_acc[...] = jnp.zeros_like(gc_acc)

        agg = jnp.concatenate([aggc_ref[c] for c in range(4)], axis=1)
        h = _dotT(agg * _inv_from_cnt(cnt_ref), wl_ref[...]) \
            + bl_ref[...] + r_ref[...]

        gids = jax.lax.broadcasted_iota(jnp.int32, (1, N_GRAPHS), 1)
        onehot = (b_ref[...] == gids).astype(F32)
        ps_acc[...] += jax.lax.dot_general(
            onehot, h, dimension_numbers=(((0,), (0,)), ((), ())),
            precision=_PREC, preferred_element_type=F32)
        gc_acc[...] += jnp.sum(onehot, axis=0)[:, None]

        @pl.when(i == _NBLK - 1)
        def _():
            pooled = ps_acc[...] / jnp.maximum(gc_acc[...], 1.0)
            out_ref[...] = _dotT(pooled, wlin_ref[...]) + blin_ref[...]

    return pl.pallas_call(
        body,
        grid=(_NBLK,),
        in_specs=[
            pl.BlockSpec((4, _BLK, CW), lambda i: (0, i, 0)),
            pl.BlockSpec((2, _BLK, 16), lambda i: (0, i, 0)),
            pl.BlockSpec((_BLK, 512), lambda i: (i, 0)),
            pl.BlockSpec((512, 512), lambda i: (0, 0)),
            pl.BlockSpec((1, 512), lambda i: (0, 0)),
            pl.BlockSpec((_BLK, 1), lambda i: (i, 0)),
            pl.BlockSpec((N_GRAPHS, 512), lambda i: (0, 0)),
            pl.BlockSpec((1, N_GRAPHS), lambda i: (0, 0)),
        ],
        out_specs=pl.BlockSpec((N_GRAPHS, N_GRAPHS), lambda i: (0, 0)),
        out_shape=jax.ShapeDtypeStruct((N_GRAPHS, N_GRAPHS), F32),
        scratch_shapes=[
            pltpu.VMEM((N_GRAPHS, 512), F32),
            pltpu.VMEM((N_GRAPHS, 1), F32),
        ],
    )(aggc, cnt, r, wl, bl, batch2d, wlin, blin)


def kernel(x, edge_index, batch, W1l, b1l, W1r, W2l, b2l, W2r, W3l, b3l, W3r,
           Wlin, blin):
    n_edges = edge_index.shape[1]
    eb = 192
    e_tile = n_edges // N_TILES
    nb = -(-e_tile // eb)
    pad = nb * eb - e_tile
    ebc, nbc = 500, n_edges // (2 * N_TILES * 500)
    assert e_tile * N_TILES == n_edges
    assert nbc * 2 * N_TILES * ebc == n_edges

    src32 = edge_index[0].astype(jnp.int32)
    dst32 = edge_index[1].astype(jnp.int32)
    pad_src = jnp.broadcast_to(jnp.arange(pad, dtype=jnp.int32) % 8,
                               (N_TILES, pad))
    pad_dst = pad_src + N_NODES
    src_p = jnp.concatenate(
        [src32.reshape(N_TILES, e_tile), pad_src], axis=1).reshape(
            N_TILES, nb, 1, eb)
    dst_p = jnp.concatenate(
        [dst32.reshape(N_TILES, e_tile), pad_dst], axis=1).reshape(
            N_TILES, nb, 1, eb)
    idx_p = jnp.concatenate([src_p, dst_p], axis=2)
    dstc = dst32.reshape(2, N_TILES, nbc, ebc)
    batch2d = batch.astype(jnp.int32).reshape(N_NODES, 1)
    zrow = jnp.zeros((ROWS_MAIN, CW), F32)
    z16 = jnp.zeros((ROWS_MAIN, 16), F32)
    ones16 = jnp.ones((ebc, 16), F32)
    b1l2 = b1l.reshape(1, 512)
    b2l2 = b2l.reshape(1, 512)
    b3l2 = b3l.reshape(1, 512)
    blin2 = blin.reshape(1, N_GRAPHS)

    sc_first = _make_sc_segsum(2, nb, eb)
    sc_hidden = _make_sc_segsum(4, nb, eb)
    sc_cnt = _make_sc_cnt(nbc, ebc)

    cnt = sc_cnt(dstc, ones16, z16)
    xck, r1 = _tc_prep(x, W1r)
    agg1c = sc_first(xck, idx_p, zrow)
    h1ck, r2 = _tc_layer(agg1c, cnt, r1, W1l, b1l2, W2r, relu=True)
    agg2c = sc_hidden(h1ck, idx_p, zrow)
    h2ck, r3 = _tc_layer(agg2c, cnt, r2, W2l, b2l2, W3r, relu=True)
    agg3c = sc_hidden(h2ck, idx_p, zrow)
    out = _tc_final(agg3c, cnt, r3, W3l, b3l2, batch2d, Wlin, blin2)
    return out


# speedup vs baseline: 1.0267x; 1.0027x over previous
import jax
import jax.numpy as jnp
from jax import lax
from jax.experimental import pallas as pl
from jax.experimental.pallas import tpu as pltpu
from jax.experimental.pallas import tpu_sc as plsc

N_NODES = 10000
N_GRAPHS = 64
N_TILES = 16
ROWS_MAIN = 632
ROWS_LAST = N_NODES - (N_TILES - 1) * ROWS_MAIN
CW = 128
_ROWS_DEPTH = 3
_EDGE_BATCH = 128
F32 = jnp.float32

_MESH = plsc.VectorSubcoreMesh(core_axis_name="c", subcore_axis_name="s")
_SC_PARAMS = pltpu.CompilerParams(use_tc_tiling_on_sc=False)

_PREC = jax.lax.Precision.DEFAULT


def _dotT(a, b):
    return jax.lax.dot_general(
        a, b, dimension_numbers=(((1,), (1,)), ((), ())),
        precision=_PREC, preferred_element_type=F32)


def _make_sc_segsum(n_chunks: int, n_batches: int, edge_batch: int):
    cc = n_chunks // 2
    nb = n_batches

    nd = _ROWS_DEPTH
    ni = nd + 1
    scratch = (
        [pltpu.VMEM((2, edge_batch), jnp.int32)] * ni
        + [pltpu.VMEM((edge_batch, CW), F32)] * nd
        + [pltpu.VMEM_SHARED((N_NODES + 8, CW), F32)]
        + [pltpu.SemaphoreType.DMA] * (ni + 2 * nd)
    )

    def body(table, idx_h, zrow_h, out_h, *rest):
        ibs = rest[:ni]
        rbs = rest[ni:ni + nd]
        acc = rest[ni + nd]
        sems = rest[ni + nd + 1:]
        isems = sems[:ni]
        gsems = sems[ni:ni + nd]
        ssems = sems[ni + nd:]

        cid = lax.axis_index("c")
        sid = lax.axis_index("s")

        def per_tile_rows(fn):
            @pl.when(sid < N_TILES - 1)
            def _():
                fn(sid * ROWS_MAIN, ROWS_MAIN)

            @pl.when(sid == N_TILES - 1)
            def _():
                fn((N_TILES - 1) * ROWS_MAIN, ROWS_LAST)

        def idx_load(b):
            return pltpu.async_copy(idx_h.at[sid, b], ibs[b % 3], isems[b % 3])

        for j in range(cc):
            c = cid * cc + j
            per_tile_rows(lambda r0, nr: pltpu.sync_copy(
                zrow_h.at[pl.ds(0, nr)], acc.at[pl.ds(r0, nr)]))
            plsc.subcore_barrier()

            tbl_c = table.at[c]

            def gather(b):
                return pltpu.async_copy(
                    tbl_c.at[ibs[b % ni].at[0]], rbs[b % nd], gsems[b % nd])

            def scatter(b):
                return pltpu.async_copy(
                    rbs[b % nd], acc.at[ibs[b % ni].at[1]], ssems[b % nd],
                    add=True)

            il = {b: idx_load(b) for b in range(min(ni, nb))}
            il[0].wait()
            g = {0: gather(0)}
            s = {}
            s_waited = set()
            for b in range(nb):
                g[b].wait()
                if b + 1 < nb:
                    il[b + 1].wait()
                    sb = b + 1 - nd
                    if sb in s and sb not in s_waited:
                        s[sb].wait()
                        s_waited.add(sb)
                        nxt = sb + ni
                        if ni <= nxt < nb:
                            il[nxt] = idx_load(nxt)
                    g[b + 1] = gather(b + 1)
                s[b] = scatter(b)
            for b in range(nb):
                if b in s and b not in s_waited:
                    s[b].wait()
            plsc.subcore_barrier()

            per_tile_rows(lambda r0, nr: pltpu.sync_copy(
                acc.at[pl.ds(r0, nr)], out_h.at[c, pl.ds(r0, nr)]))

    return pl.kernel(
        body,
        out_type=jax.ShapeDtypeStruct((n_chunks, N_NODES, CW), F32),
        mesh=_MESH, scratch_types=scratch, compiler_params=_SC_PARAMS)


def _make_sc_cnt(n_batches: int, edge_batch: int):
    scratch = [
        pltpu.VMEM((n_batches, edge_batch), jnp.int32),
        pltpu.VMEM((edge_batch, 16), F32),
        pltpu.VMEM_SHARED((N_NODES, 16), F32),
        pltpu.SemaphoreType.DMA,
    ]

    def body(dst_h, ones_h, z16_h, cnt_h, dst_v, ones_v, cntacc, csem):
        cid = lax.axis_index("c")
        sid = lax.axis_index("s")

        def per_tile_rows(fn):
            @pl.when(sid < N_TILES - 1)
            def _():
                fn(sid * ROWS_MAIN, ROWS_MAIN)

            @pl.when(sid == N_TILES - 1)
            def _():
                fn((N_TILES - 1) * ROWS_MAIN, ROWS_LAST)

        pltpu.sync_copy(dst_h.at[cid, sid], dst_v)
        pltpu.sync_copy(ones_h, ones_v)
        per_tile_rows(lambda r0, nr: pltpu.sync_copy(
            z16_h.at[pl.ds(0, nr)], cntacc.at[pl.ds(r0, nr)]))
        plsc.subcore_barrier()
        pending = []
        for b in range(n_batches):
            pending.append(pltpu.async_copy(
                ones_v, cntacc.at[dst_v.at[b]], csem, add=True))
        for p in pending:
            p.wait()
        plsc.subcore_barrier()
        per_tile_rows(lambda r0, nr: pltpu.sync_copy(
            cntacc.at[pl.ds(r0, nr)], cnt_h.at[cid, pl.ds(r0, nr)]))

    return pl.kernel(
        body,
        out_type=jax.ShapeDtypeStruct((2, N_NODES, 16), F32),
        mesh=_MESH, scratch_types=scratch, compiler_params=_SC_PARAMS)


_BLK = 1000
_NBLK = N_NODES // _BLK


def _inv_from_cnt(cnt_ref):
    c = cnt_ref[0, :, :1] + cnt_ref[1, :, :1]
    return 1.0 / jnp.maximum(c, 1.0)


def _tc_prep(x, w1r):
    d_in = x.shape[1]
    nc = d_in // CW

    def body(x_ref, w_ref, xck_ref, r_ref):
        xb = x_ref[...]
        r_ref[...] = _dotT(xb, w_ref[...])
        for c in range(nc):
            xck_ref[c] = xb[:, c * CW:(c + 1) * CW]

    return pl.pallas_call(
        body,
        grid=(_NBLK,),
        in_specs=[
            pl.BlockSpec((_BLK, d_in), lambda i: (i, 0)),
            pl.BlockSpec((512, d_in), lambda i: (0, 0)),
        ],
        out_specs=[
            pl.BlockSpec((nc, _BLK, CW), lambda i: (0, i, 0)),
            pl.BlockSpec((_BLK, 512), lambda i: (i, 0)),
        ],
        out_shape=[
            jax.ShapeDtypeStruct((nc, N_NODES, CW), F32),
            jax.ShapeDtypeStruct((N_NODES, 512), F32),
        ],
    )(x, w1r)


def _tc_layer(aggc, cnt, r, wl, bl, w_next, relu: bool):
    nc_in = aggc.shape[0]
    d_in = nc_in * CW

    def body(aggc_ref, cnt_ref, r_ref, wl_ref, bl_ref, wn_ref,
             hck_ref, rn_ref):
        agg = jnp.concatenate([aggc_ref[c] for c in range(nc_in)], axis=1)
        h = _dotT(agg * _inv_from_cnt(cnt_ref), wl_ref[...]) \
            + bl_ref[...] + r_ref[...]
        if relu:
            h = jnp.maximum(h, 0.0)
        rn_ref[...] = _dotT(h, wn_ref[...])
        for c in range(4):
            hck_ref[c] = h[:, c * CW:(c + 1) * CW]

    return pl.pallas_call(
        body,
        grid=(_NBLK,),
        in_specs=[
            pl.BlockSpec((nc_in, _BLK, CW), lambda i: (0, i, 0)),
            pl.BlockSpec((2, _BLK, 16), lambda i: (0, i, 0)),
            pl.BlockSpec((_BLK, 512), lambda i: (i, 0)),
            pl.BlockSpec((512, d_in), lambda i: (0, 0)),
            pl.BlockSpec((1, 512), lambda i: (0, 0)),
            pl.BlockSpec((512, 512), lambda i: (0, 0)),
        ],
        out_specs=[
            pl.BlockSpec((4, _BLK, CW), lambda i: (0, i, 0)),
            pl.BlockSpec((_BLK, 512), lambda i: (i, 0)),
        ],
        out_shape=[
            jax.ShapeDtypeStruct((4, N_NODES, CW), F32),
            jax.ShapeDtypeStruct((N_NODES, 512), F32),
        ],
    )(aggc, cnt, r, wl, bl, w_next)


def _tc_final(aggc, cnt, r, wl, bl, batch2d, wlin, blin):
    def body(aggc_ref, cnt_ref, r_ref, wl_ref, bl_ref, b_ref,
             wlin_ref, blin_ref, out_ref, ps_acc, gc_acc):
        i = pl.program_id(0)

        @pl.when(i == 0)
        def _():
            ps_acc[...] = jnp.zeros_like(ps_acc)
            gc_acc[...] = jnp.zeros_like(gc_acc)

        agg = jnp.concatenate([aggc_ref[c] for c in range(4)], axis=1)
        h = _dotT(agg * _inv_from_cnt(cnt_ref), wl_ref[...]) \
            + bl_ref[...] + r_ref[...]

        gids = jax.lax.broadcasted_iota(jnp.int32, (1, N_GRAPHS), 1)
        onehot = (b_ref[...] == gids).astype(F32)
        ps_acc[...] += jax.lax.dot_general(
            onehot, h, dimension_numbers=(((0,), (0,)), ((), ())),
            precision=_PREC, preferred_element_type=F32)
        gc_acc[...] += jnp.sum(onehot, axis=0)[:, None]

        @pl.when(i == _NBLK - 1)
        def _():
            pooled = ps_acc[...] / jnp.maximum(gc_acc[...], 1.0)
            out_ref[...] = _dotT(pooled, wlin_ref[...]) + blin_ref[...]

    return pl.pallas_call(
        body,
        grid=(_NBLK,),
        in_specs=[
            pl.BlockSpec((4, _BLK, CW), lambda i: (0, i, 0)),
            pl.BlockSpec((2, _BLK, 16), lambda i: (0, i, 0)),
            pl.BlockSpec((_BLK, 512), lambda i: (i, 0)),
            pl.BlockSpec((512, 512), lambda i: (0, 0)),
            pl.BlockSpec((1, 512), lambda i: (0, 0)),
            pl.BlockSpec((_BLK, 1), lambda i: (i, 0)),
            pl.BlockSpec((N_GRAPHS, 512), lambda i: (0, 0)),
            pl.BlockSpec((1, N_GRAPHS), lambda i: (0, 0)),
        ],
        out_specs=pl.BlockSpec((N_GRAPHS, N_GRAPHS), lambda i: (0, 0)),
        out_shape=jax.ShapeDtypeStruct((N_GRAPHS, N_GRAPHS), F32),
        scratch_shapes=[
            pltpu.VMEM((N_GRAPHS, 512), F32),
            pltpu.VMEM((N_GRAPHS, 1), F32),
        ],
    )(aggc, cnt, r, wl, bl, batch2d, wlin, blin)


def kernel(x, edge_index, batch, W1l, b1l, W1r, W2l, b2l, W2r, W3l, b3l, W3r,
           Wlin, blin):
    n_edges = edge_index.shape[1]
    eb = _EDGE_BATCH
    e_tile = n_edges // N_TILES
    nb = -(-e_tile // eb)
    pad = nb * eb - e_tile
    ebc, nbc = 500, n_edges // (2 * N_TILES * 500)
    assert e_tile * N_TILES == n_edges
    assert nbc * 2 * N_TILES * ebc == n_edges

    src32 = edge_index[0].astype(jnp.int32)
    dst32 = edge_index[1].astype(jnp.int32)
    pad_src = jnp.broadcast_to(jnp.arange(pad, dtype=jnp.int32) % 8,
                               (N_TILES, pad))
    pad_dst = pad_src + N_NODES
    src_p = jnp.concatenate(
        [src32.reshape(N_TILES, e_tile), pad_src], axis=1).reshape(
            N_TILES, nb, 1, eb)
    dst_p = jnp.concatenate(
        [dst32.reshape(N_TILES, e_tile), pad_dst], axis=1).reshape(
            N_TILES, nb, 1, eb)
    idx_p = jnp.concatenate([src_p, dst_p], axis=2)
    dstc = dst32.reshape(2, N_TILES, nbc, ebc)
    batch2d = batch.astype(jnp.int32).reshape(N_NODES, 1)
    zrow = jnp.zeros((ROWS_MAIN, CW), F32)
    z16 = jnp.zeros((ROWS_MAIN, 16), F32)
    ones16 = jnp.ones((ebc, 16), F32)
    b1l2 = b1l.reshape(1, 512)
    b2l2 = b2l.reshape(1, 512)
    b3l2 = b3l.reshape(1, 512)
    blin2 = blin.reshape(1, N_GRAPHS)

    sc_first = _make_sc_segsum(2, nb, eb)
    sc_hidden = _make_sc_segsum(4, nb, eb)
    sc_cnt = _make_sc_cnt(nbc, ebc)

    cnt = sc_cnt(dstc, ones16, z16)
    xck, r1 = _tc_prep(x, W1r)
    agg1c = sc_first(xck, idx_p, zrow)
    h1ck, r2 = _tc_layer(agg1c, cnt, r1, W1l, b1l2, W2r, relu=True)
    agg2c = sc_hidden(h1ck, idx_p, zrow)
    h2ck, r3 = _tc_layer(agg2c, cnt, r2, W2l, b2l2, W3r, relu=True)
    agg3c = sc_hidden(h2ck, idx_p, zrow)
    out = _tc_final(agg3c, cnt, r3, W3l, b3l2, batch2d, Wlin, blin2)
    return out
